# baseline (speedup 1.0000x reference)
import functools

import jax
import jax.numpy as jnp
import numpy as np
from jax import lax
from jax.experimental import pallas as pl
from jax.experimental.pallas import tpu as pltpu
from jax.experimental.pallas import tpu_sc as plsc

DIM = 128
HALF = 64
MAX_POS = 8192
BASE = 10000.0

NC = 2
NS = 16
NW = NC * NS
B = 4 * 4096
PER_W = B // NW
CHUNK = 128
NCHUNK = PER_W // CHUNK

_inv_freq = 1.0 / (BASE ** (np.arange(0, DIM, 2, dtype=np.float32) / DIM))
_freqs = np.arange(MAX_POS, dtype=np.float32)[:, None] * _inv_freq[None, :]
_COS_TAB = np.cos(_freqs, dtype=np.float32)
_SIN_TAB = np.sin(_freqs, dtype=np.float32)

_mesh = plsc.VectorSubcoreMesh(core_axis_name="c", subcore_axis_name="s")


@functools.partial(
    pl.kernel,
    out_type=(
        jax.ShapeDtypeStruct((B, DIM), jnp.float32),
        jax.ShapeDtypeStruct((B, DIM), jnp.float32),
    ),
    mesh=_mesh,
    scratch_types=[
        pltpu.VMEM((NCHUNK, CHUNK), jnp.int32),
        pltpu.VMEM((NCHUNK, CHUNK, HALF), jnp.float32),
        pltpu.VMEM((NCHUNK, CHUNK, HALF), jnp.float32),
        [pltpu.SemaphoreType.DMA] * NCHUNK,
        pltpu.SemaphoreType.DMA,
    ],
    compiler_params=pltpu.CompilerParams(use_tc_tiling_on_sc=False),
)
def _rope_gather(cos_hbm, sin_hbm, ids_hbm, cos_out, sin_out,
                 idx_v, cos_v, sin_v, sems_g, sem_w):
    wid = lax.axis_index("s") * NC + lax.axis_index("c")
    base = wid * PER_W
    pltpu.sync_copy(ids_hbm.at[pl.ds(wid * NCHUNK, NCHUNK)], idx_v)
    gathers = []
    for j in range(NCHUNK):
        idx_row = idx_v.at[j]
        gathers.append(
            (pltpu.async_copy(cos_hbm.at[idx_row], cos_v.at[j], sems_g[j]),
             pltpu.async_copy(sin_hbm.at[idx_row], sin_v.at[j], sems_g[j])))
    writes = []
    for j in range(NCHUNK):
        gc, gs = gathers[j]
        gc.wait()
        gs.wait()
        rb = base + j * CHUNK
        writes.append(pltpu.async_copy(
            cos_v.at[j], cos_out.at[pl.ds(rb, CHUNK), pl.ds(0, HALF)], sem_w))
        writes.append(pltpu.async_copy(
            cos_v.at[j], cos_out.at[pl.ds(rb, CHUNK), pl.ds(HALF, HALF)], sem_w))
        writes.append(pltpu.async_copy(
            sin_v.at[j], sin_out.at[pl.ds(rb, CHUNK), pl.ds(0, HALF)], sem_w))
        writes.append(pltpu.async_copy(
            sin_v.at[j], sin_out.at[pl.ds(rb, CHUNK), pl.ds(HALF, HALF)], sem_w))
    for w in writes:
        w.wait()


def kernel(x, position_ids):
    bsz, seq = position_ids.shape
    cos_t = jnp.asarray(_COS_TAB)
    sin_t = jnp.asarray(_SIN_TAB)
    ids = position_ids.reshape(NW * NCHUNK, CHUNK).astype(jnp.int32)
    cos_f, sin_f = _rope_gather(cos_t, sin_t, ids)
    return cos_f.reshape(bsz, seq, DIM), sin_f.reshape(bsz, seq, DIM)


# speedup vs baseline: 1.0078x; 1.0078x over previous
import functools

import jax
import jax.numpy as jnp
import numpy as np
from jax import lax
from jax.experimental import pallas as pl
from jax.experimental.pallas import tpu as pltpu
from jax.experimental.pallas import tpu_sc as plsc

DIM = 128
HALF = 64
MAX_POS = 8192
BASE = 10000.0

NC = 2
NS = 16
NW = NC * NS
B = 4 * 4096
PER_W = B // NW
CHUNK = 128
NCHUNK = PER_W // CHUNK

_inv_freq = 1.0 / (BASE ** (np.arange(0, DIM, 2, dtype=np.float32) / DIM))
_freqs = np.arange(MAX_POS, dtype=np.float32)[:, None] * _inv_freq[None, :]
_COS_TAB = np.cos(_freqs, dtype=np.float32)
_SIN_TAB = np.sin(_freqs, dtype=np.float32)

_mesh = plsc.VectorSubcoreMesh(core_axis_name="c", subcore_axis_name="s")


@functools.partial(
    pl.kernel,
    out_type=(
        jax.ShapeDtypeStruct((B, DIM), jnp.float32),
        jax.ShapeDtypeStruct((B, DIM), jnp.float32),
    ),
    mesh=_mesh,
    scratch_types=[
        pltpu.VMEM((PER_W,), jnp.int32),
        pltpu.VMEM((PER_W, HALF), jnp.float32),
        pltpu.VMEM((PER_W, HALF), jnp.float32),
        pltpu.SemaphoreType.DMA,
        pltpu.SemaphoreType.DMA,
        pltpu.SemaphoreType.DMA,
    ],
    compiler_params=pltpu.CompilerParams(use_tc_tiling_on_sc=False),
)
def _rope_gather(cos_hbm, sin_hbm, ids_hbm, cos_out, sin_out,
                 idx_v, cos_v, sin_v, sem_c, sem_s, sem_w):
    wid = lax.axis_index("s") * NC + lax.axis_index("c")
    base = wid * PER_W
    pltpu.sync_copy(ids_hbm.at[pl.ds(base, PER_W)], idx_v)
    gc = pltpu.async_copy(cos_hbm.at[idx_v], cos_v, sem_c)
    gs = pltpu.async_copy(sin_hbm.at[idx_v], sin_v, sem_s)
    writes = []
    gc.wait()
    writes.append(pltpu.async_copy(
        cos_v, cos_out.at[pl.ds(base, PER_W), pl.ds(0, HALF)], sem_w))
    writes.append(pltpu.async_copy(
        cos_v, cos_out.at[pl.ds(base, PER_W), pl.ds(HALF, HALF)], sem_w))
    gs.wait()
    writes.append(pltpu.async_copy(
        sin_v, sin_out.at[pl.ds(base, PER_W), pl.ds(0, HALF)], sem_w))
    writes.append(pltpu.async_copy(
        sin_v, sin_out.at[pl.ds(base, PER_W), pl.ds(HALF, HALF)], sem_w))
    for w in writes:
        w.wait()


def kernel(x, position_ids):
    bsz, seq = position_ids.shape
    cos_t = jnp.asarray(_COS_TAB)
    sin_t = jnp.asarray(_SIN_TAB)
    ids = position_ids.reshape(-1).astype(jnp.int32)
    cos_f, sin_f = _rope_gather(cos_t, sin_t, ids)
    return cos_f.reshape(bsz, seq, DIM), sin_f.reshape(bsz, seq, DIM)


# speedup vs baseline: 1.0139x; 1.0060x over previous
import functools

import jax
import jax.numpy as jnp
import numpy as np
from jax import lax
from jax.experimental import pallas as pl
from jax.experimental.pallas import tpu as pltpu
from jax.experimental.pallas import tpu_sc as plsc

DIM = 128
HALF = 64
MAX_POS = 8192
BASE = 10000.0

NC = 2
NS = 16
NW = NC * NS
B = 4 * 4096
PER_W = B // NW
CHUNK = 128
NCHUNK = PER_W // CHUNK

_inv_freq = 1.0 / (BASE ** (np.arange(0, DIM, 2, dtype=np.float32) / DIM))
_freqs = np.arange(MAX_POS, dtype=np.float32)[:, None] * _inv_freq[None, :]
_COS_TAB = np.cos(_freqs, dtype=np.float32)
_SIN_TAB = np.sin(_freqs, dtype=np.float32)

_mesh = plsc.VectorSubcoreMesh(core_axis_name="c", subcore_axis_name="s")


@functools.partial(
    pl.kernel,
    out_type=(
        jax.ShapeDtypeStruct((B, DIM), jnp.float32),
        jax.ShapeDtypeStruct((B, DIM), jnp.float32),
    ),
    mesh=_mesh,
    scratch_types=[
        pltpu.VMEM((PER_W,), jnp.int32),
        pltpu.VMEM((PER_W, HALF), jnp.float32),
        pltpu.VMEM((PER_W, HALF), jnp.float32),
        pltpu.SemaphoreType.DMA,
        pltpu.SemaphoreType.DMA,
        pltpu.SemaphoreType.DMA,
    ],
    compiler_params=pltpu.CompilerParams(
        use_tc_tiling_on_sc=False,
        disable_bounds_checks=True,
        disable_semaphore_checks=True,
        skip_device_barrier=True,
    ),
)
def _rope_gather(cos_hbm, sin_hbm, ids_hbm, cos_out, sin_out,
                 idx_v, cos_v, sin_v, sem_c, sem_s, sem_w):
    wid = lax.axis_index("s") * NC + lax.axis_index("c")
    base = wid * PER_W
    pltpu.sync_copy(ids_hbm.at[pl.ds(base, PER_W)], idx_v)
    gc = pltpu.async_copy(cos_hbm.at[idx_v], cos_v, sem_c)
    gs = pltpu.async_copy(sin_hbm.at[idx_v], sin_v, sem_s)
    writes = []
    gc.wait()
    writes.append(pltpu.async_copy(
        cos_v, cos_out.at[pl.ds(base, PER_W), pl.ds(0, HALF)], sem_w))
    writes.append(pltpu.async_copy(
        cos_v, cos_out.at[pl.ds(base, PER_W), pl.ds(HALF, HALF)], sem_w))
    gs.wait()
    writes.append(pltpu.async_copy(
        sin_v, sin_out.at[pl.ds(base, PER_W), pl.ds(0, HALF)], sem_w))
    writes.append(pltpu.async_copy(
        sin_v, sin_out.at[pl.ds(base, PER_W), pl.ds(HALF, HALF)], sem_w))
    for w in writes:
        w.wait()


def kernel(x, position_ids):
    bsz, seq = position_ids.shape
    cos_t = jnp.asarray(_COS_TAB)
    sin_t = jnp.asarray(_SIN_TAB)
    ids = position_ids.reshape(-1).astype(jnp.int32)
    cos_f, sin_f = _rope_gather(cos_t, sin_t, ids)
    return cos_f.reshape(bsz, seq, DIM), sin_f.reshape(bsz, seq, DIM)
